# baseline (speedup 1.0000x reference)
import functools

import numpy as np
import jax
import jax.numpy as jnp
from jax import lax
from jax.experimental import pallas as pl
from jax.experimental.pallas import tpu as pltpu
from jax.experimental.pallas import tpu_sc as plsc

_VOCAB = 1000000
_D = 64
_DP = 128
_NSEG = 3
_SEQ = 200
_BATCH = 1024
_B = _BATCH * _SEQ
_NC, _NS = 2, 16
_NW = _NC * _NS
_BPW = _B // _NW
_CH = 320
_NCHUNK = _BPW // _CH
_LANES = 16


def _pe_table():
    position = np.arange(_SEQ, dtype=np.float32)[:, None]
    div_term = np.exp(
        np.arange(0, _D, 2, dtype=np.float32) * -(np.log(10000.0) / _D))
    pe = np.zeros((_SEQ, _D), dtype=np.float32)
    pe[:, 0::2] = np.sin(position * div_term)
    pe[:, 1::2] = np.cos(position * div_term)
    return pe


_PE = _pe_table()


def _body(seq_hbm, seg_hbm, tok_hbm, comb_hbm, out_hbm,
          idx_v0, idx_v1, seg_v0, seg_v1, cidx_v0, cidx_v1,
          rows_t0, rows_t1, rows_c0, rows_c1,
          sem_t0, sem_t1, sem_c0, sem_c1, sem_o0, sem_o1):
    idx_v = (idx_v0, idx_v1)
    seg_v = (seg_v0, seg_v1)
    cidx_v = (cidx_v0, cidx_v1)
    rows_t = (rows_t0, rows_t1)
    rows_c = (rows_c0, rows_c1)
    sem_t = (sem_t0, sem_t1)
    sem_c = (sem_c0, sem_c1)
    sem_o = (sem_o0, sem_o1)

    wid = lax.axis_index("s") * _NC + lax.axis_index("c")
    base = wid * _BPW

    def chunk_off(c):
        return pl.multiple_of(base + c * _CH, 8)

    def prep(c, s):
        cb = chunk_off(c)
        pltpu.sync_copy(seq_hbm.at[pl.ds(cb, _CH)], idx_v[s])
        pltpu.sync_copy(seg_hbm.at[pl.ds(cb, _CH)], seg_v[s])
        for j in range(_CH // _LANES):
            sl = pl.ds(j * _LANES, _LANES)
            pos = jnp.remainder(
                c * _CH + j * _LANES + lax.iota(jnp.int32, _LANES), _SEQ)
            cidx_v[s][sl] = seg_v[s][sl] * _SEQ + pos

    def issue_gathers(s):
        pltpu.async_copy(tok_hbm.at[idx_v[s]], rows_t[s], sem_t[s])
        pltpu.async_copy(comb_hbm.at[cidx_v[s]], rows_c[s], sem_c[s])

    def wait_gathers(s):
        pltpu.make_async_copy(tok_hbm.at[idx_v[s]], rows_t[s], sem_t[s]).wait()
        pltpu.make_async_copy(
            comb_hbm.at[cidx_v[s]], rows_c[s], sem_c[s]).wait()

    def out_desc(c, s):
        return pltpu.make_async_copy(
            rows_c[s], out_hbm.at[pl.ds(chunk_off(c), _CH)], sem_o[s])

    for s in range(2):
        prep(s, s)
        issue_gathers(s)

    def step(i, carry):
        for s in range(2):
            c = 2 * i + s
            wait_gathers(s)

            def add_body(t, acc):
                for j in range(_D // _LANES):
                    sl = pl.ds(j * _LANES, _LANES)
                    rows_c[s][t, sl] = rows_c[s][t, sl] + rows_t[s][t, sl]
                return acc

            lax.fori_loop(0, _CH, add_body, 0, unroll=4)
            out_desc(c, s).start()

            @pl.when(c + 2 < _NCHUNK)
            def _():
                prep(c + 2, s)
                out_desc(c, s).wait()
                issue_gathers(s)

        return carry

    lax.fori_loop(0, _NCHUNK // 2, step, 0)

    for s in range(2):
        out_desc(_NCHUNK - 2 + s, s).wait()


_lookup = functools.partial(
    pl.kernel,
    out_type=jax.ShapeDtypeStruct((_B, _D), jnp.float32),
    mesh=plsc.VectorSubcoreMesh(
        core_axis_name="c", subcore_axis_name="s",
        num_cores=_NC, num_subcores=_NS),
    scratch_types=[
        pltpu.VMEM((_CH,), jnp.int32),
        pltpu.VMEM((_CH,), jnp.int32),
        pltpu.VMEM((_CH,), jnp.int32),
        pltpu.VMEM((_CH,), jnp.int32),
        pltpu.VMEM((_CH,), jnp.int32),
        pltpu.VMEM((_CH,), jnp.int32),
        pltpu.VMEM((_CH, _DP), jnp.float32),
        pltpu.VMEM((_CH, _DP), jnp.float32),
        pltpu.VMEM((_CH, _D), jnp.float32),
        pltpu.VMEM((_CH, _D), jnp.float32),
        pltpu.SemaphoreType.DMA,
        pltpu.SemaphoreType.DMA,
        pltpu.SemaphoreType.DMA,
        pltpu.SemaphoreType.DMA,
        pltpu.SemaphoreType.DMA,
        pltpu.SemaphoreType.DMA,
    ],
    compiler_params=pltpu.CompilerParams(use_tc_tiling_on_sc=False),
)(_body)


@jax.jit
def kernel(sequence, segment_labels, tok_table, seg_table):
    comb = (seg_table[:, None, :] + jnp.asarray(_PE)[None, :, :])
    comb = comb.reshape(_NSEG * _SEQ, _D)
    tok128 = jnp.concatenate(
        [tok_table, jnp.zeros((_VOCAB, _DP - _D), jnp.float32)], axis=1)
    seq_flat = sequence.reshape(_B)
    seg_flat = segment_labels.reshape(_B)
    out = _lookup(seq_flat, seg_flat, tok128, comb)
    return out.reshape(_BATCH, _SEQ, _D)


# speedup vs baseline: 1.2984x; 1.2984x over previous
import functools

import numpy as np
import jax
import jax.numpy as jnp
from jax import lax
from jax.experimental import pallas as pl
from jax.experimental.pallas import tpu as pltpu
from jax.experimental.pallas import tpu_sc as plsc

_VOCAB = 1000000
_D = 64
_DP = 128
_NSEG = 3
_SEQ = 200
_BATCH = 1024
_B = _BATCH * _SEQ
_NC, _NS = 2, 16
_NW = _NC * _NS
_BPW = _B // _NW
_CH = 400
_NCHUNK = _BPW // _CH
_LANES = 16
_CB = 4096


def _pad_body(tok_t_ref, out_ref):
    out_ref[:, 0:_D] = tok_t_ref[...].T


_pad_rows = pl.pallas_call(
    _pad_body,
    out_shape=jax.ShapeDtypeStruct((_VOCAB, _DP), jnp.float32),
    grid=((_VOCAB + _CB - 1) // _CB,),
    in_specs=[pl.BlockSpec((_D, _CB), lambda i: (0, i))],
    out_specs=pl.BlockSpec((_CB, _DP), lambda i: (i, 0)),
)


def _pe_table():
    position = np.arange(_SEQ, dtype=np.float32)[:, None]
    div_term = np.exp(
        np.arange(0, _D, 2, dtype=np.float32) * -(np.log(10000.0) / _D))
    pe = np.zeros((_SEQ, _D), dtype=np.float32)
    pe[:, 0::2] = np.sin(position * div_term)
    pe[:, 1::2] = np.cos(position * div_term)
    return pe


_PE = _pe_table()


def _body(seq_hbm, seg_hbm, tok_hbm, comb_hbm, out_hbm,
          idx_v0, idx_v1, seg_v0, seg_v1, cidx_v0, cidx_v1,
          rows_t0, rows_t1, rows_c0, rows_c1,
          sem_t0, sem_t1, sem_c0, sem_c1, sem_o0, sem_o1):
    idx_v = (idx_v0, idx_v1)
    seg_v = (seg_v0, seg_v1)
    cidx_v = (cidx_v0, cidx_v1)
    rows_t = (rows_t0, rows_t1)
    rows_c = (rows_c0, rows_c1)
    sem_t = (sem_t0, sem_t1)
    sem_c = (sem_c0, sem_c1)
    sem_o = (sem_o0, sem_o1)

    wid = lax.axis_index("s") * _NC + lax.axis_index("c")
    base = wid * _BPW

    def chunk_off(c):
        return pl.multiple_of(base + c * _CH, 8)

    def prep(c, s):
        cb = chunk_off(c)
        pltpu.sync_copy(seq_hbm.at[pl.ds(cb, _CH)], idx_v[s])
        pltpu.sync_copy(seg_hbm.at[pl.ds(cb, _CH)], seg_v[s])
        for j in range(_CH // _LANES):
            sl = pl.ds(j * _LANES, _LANES)
            pos = jnp.remainder(
                c * _CH + j * _LANES + lax.iota(jnp.int32, _LANES), _SEQ)
            cidx_v[s][sl] = seg_v[s][sl] * _SEQ + pos
            idx_v[s][sl] = idx_v[s][sl] * 2

    def issue_gathers(s):
        pltpu.async_copy(tok_hbm.at[idx_v[s]], rows_t[s], sem_t[s])
        pltpu.async_copy(comb_hbm.at[cidx_v[s]], rows_c[s], sem_c[s])

    def wait_gathers(s):
        pltpu.make_async_copy(tok_hbm.at[idx_v[s]], rows_t[s], sem_t[s]).wait()
        pltpu.make_async_copy(
            comb_hbm.at[cidx_v[s]], rows_c[s], sem_c[s]).wait()

    def out_desc(c, s):
        return pltpu.make_async_copy(
            rows_c[s], out_hbm.at[pl.ds(chunk_off(c), _CH)], sem_o[s])

    for s in range(2):
        prep(s, s)
        issue_gathers(s)

    def step(i, carry):
        for s in range(2):
            c = 2 * i + s
            wait_gathers(s)

            def add_body(t, acc):
                for j in range(_D // _LANES):
                    sl = pl.ds(j * _LANES, _LANES)
                    rows_c[s][t, sl] = rows_c[s][t, sl] + rows_t[s][t, sl]
                return acc

            lax.fori_loop(0, _CH, add_body, 0, unroll=4)
            out_desc(c, s).start()

            @pl.when(c + 2 < _NCHUNK)
            def _():
                prep(c + 2, s)
                out_desc(c, s).wait()
                issue_gathers(s)

        return carry

    lax.fori_loop(0, _NCHUNK // 2, step, 0)

    for s in range(2):
        out_desc(_NCHUNK - 2 + s, s).wait()


_lookup = functools.partial(
    pl.kernel,
    out_type=jax.ShapeDtypeStruct((_B, _D), jnp.float32),
    mesh=plsc.VectorSubcoreMesh(
        core_axis_name="c", subcore_axis_name="s",
        num_cores=_NC, num_subcores=_NS),
    scratch_types=[
        pltpu.VMEM((_CH,), jnp.int32),
        pltpu.VMEM((_CH,), jnp.int32),
        pltpu.VMEM((_CH,), jnp.int32),
        pltpu.VMEM((_CH,), jnp.int32),
        pltpu.VMEM((_CH,), jnp.int32),
        pltpu.VMEM((_CH,), jnp.int32),
        pltpu.VMEM((_CH, _D), jnp.float32),
        pltpu.VMEM((_CH, _D), jnp.float32),
        pltpu.VMEM((_CH, _D), jnp.float32),
        pltpu.VMEM((_CH, _D), jnp.float32),
        pltpu.SemaphoreType.DMA,
        pltpu.SemaphoreType.DMA,
        pltpu.SemaphoreType.DMA,
        pltpu.SemaphoreType.DMA,
        pltpu.SemaphoreType.DMA,
        pltpu.SemaphoreType.DMA,
    ],
    compiler_params=pltpu.CompilerParams(use_tc_tiling_on_sc=False),
)(_body)


@jax.jit
def kernel(sequence, segment_labels, tok_table, seg_table):
    comb = (seg_table[:, None, :] + jnp.asarray(_PE)[None, :, :])
    comb = comb.reshape(_NSEG * _SEQ, _D)
    tok128 = _pad_rows(tok_table.T)
    tok2m = tok128.reshape(2 * _VOCAB, _D)
    seq_flat = sequence.reshape(_B)
    seg_flat = segment_labels.reshape(_B)
    out = _lookup(seq_flat, seg_flat, tok2m, comb)
    return out.reshape(_BATCH, _SEQ, _D)


# speedup vs baseline: 1.4685x; 1.1310x over previous
import functools

import numpy as np
import jax
import jax.numpy as jnp
from jax import lax
from jax.experimental import pallas as pl
from jax.experimental.pallas import tpu as pltpu
from jax.experimental.pallas import tpu_sc as plsc

_VOCAB = 1000000
_D = 64
_DP = 128
_NSEG = 3
_SEQ = 200
_BATCH = 1024
_B = _BATCH * _SEQ
_NC, _NS = 2, 16
_NW = _NC * _NS
_BPW = _B // _NW
_CH = 400
_NCHUNK = _BPW // _CH
_LANES = 16
_CB = 8192


def _pad_body(tok_t_ref, out_ref):
    out_ref[:, 0:_D] = tok_t_ref[...].T


_pad_rows = pl.pallas_call(
    _pad_body,
    out_shape=jax.ShapeDtypeStruct((_VOCAB, _DP), jnp.float32),
    grid=((_VOCAB + _CB - 1) // _CB,),
    in_specs=[pl.BlockSpec((_D, _CB), lambda i: (0, i))],
    out_specs=pl.BlockSpec((_CB, _DP), lambda i: (i, 0)),
)


def _pe_table():
    position = np.arange(_SEQ, dtype=np.float32)[:, None]
    div_term = np.exp(
        np.arange(0, _D, 2, dtype=np.float32) * -(np.log(10000.0) / _D))
    pe = np.zeros((_SEQ, _D), dtype=np.float32)
    pe[:, 0::2] = np.sin(position * div_term)
    pe[:, 1::2] = np.cos(position * div_term)
    return pe


_PE = _pe_table()


def _body(seq_hbm, seg_hbm, tok_hbm, comb_hbm, out_hbm,
          idx_v0, idx_v1, seg_v0, seg_v1, cidx_v0, cidx_v1,
          rows_t0, rows_t1, rows_c0, rows_c1,
          sem_t0, sem_t1, sem_c0, sem_c1, sem_o0, sem_o1):
    idx_v = (idx_v0, idx_v1)
    seg_v = (seg_v0, seg_v1)
    cidx_v = (cidx_v0, cidx_v1)
    rows_t = (rows_t0, rows_t1)
    rows_c = (rows_c0, rows_c1)
    sem_t = (sem_t0, sem_t1)
    sem_c = (sem_c0, sem_c1)
    sem_o = (sem_o0, sem_o1)

    wid = lax.axis_index("s") * _NC + lax.axis_index("c")
    base = wid * _BPW

    def chunk_off(c):
        return pl.multiple_of(base + c * _CH, 8)

    def prep(c, s):
        cb = chunk_off(c)
        pltpu.sync_copy(seq_hbm.at[pl.ds(cb, _CH)], idx_v[s])
        pltpu.sync_copy(seg_hbm.at[pl.ds(cb, _CH)], seg_v[s])
        for j in range(_CH // _LANES):
            sl = pl.ds(j * _LANES, _LANES)
            pos = jnp.remainder(
                c * _CH + j * _LANES + lax.iota(jnp.int32, _LANES), _SEQ)
            cidx_v[s][sl] = seg_v[s][sl] * _SEQ + pos
            idx_v[s][sl] = idx_v[s][sl] * 2

    def issue_gathers(s):
        pltpu.async_copy(tok_hbm.at[idx_v[s]], rows_t[s], sem_t[s])
        pltpu.async_copy(comb_hbm.at[cidx_v[s]], rows_c[s], sem_c[s])

    def wait_gathers(s):
        pltpu.make_async_copy(tok_hbm.at[idx_v[s]], rows_t[s], sem_t[s]).wait()
        pltpu.make_async_copy(
            comb_hbm.at[cidx_v[s]], rows_c[s], sem_c[s]).wait()

    def out_desc(c, s):
        return pltpu.make_async_copy(
            rows_c[s], out_hbm.at[pl.ds(chunk_off(c), _CH)], sem_o[s])

    for s in range(2):
        prep(s, s)
        issue_gathers(s)

    def step(i, carry):
        for s in range(2):
            c = 2 * i + s
            wait_gathers(s)

            def add_body(t, acc):
                for j in range(_D // _LANES):
                    sl = pl.ds(j * _LANES, _LANES)
                    rows_c[s][t, sl] = rows_c[s][t, sl] + rows_t[s][t, sl]
                return acc

            lax.fori_loop(0, _CH, add_body, 0, unroll=4)
            out_desc(c, s).start()

            @pl.when(c + 2 < _NCHUNK)
            def _():
                prep(c + 2, s)
                out_desc(c, s).wait()
                issue_gathers(s)

        return carry

    lax.fori_loop(0, _NCHUNK // 2, step, 0)

    for s in range(2):
        out_desc(_NCHUNK - 2 + s, s).wait()


_lookup = functools.partial(
    pl.kernel,
    out_type=jax.ShapeDtypeStruct((_B, _D), jnp.float32),
    mesh=plsc.VectorSubcoreMesh(
        core_axis_name="c", subcore_axis_name="s",
        num_cores=_NC, num_subcores=_NS),
    scratch_types=[
        pltpu.VMEM((_CH,), jnp.int32),
        pltpu.VMEM((_CH,), jnp.int32),
        pltpu.VMEM((_CH,), jnp.int32),
        pltpu.VMEM((_CH,), jnp.int32),
        pltpu.VMEM((_CH,), jnp.int32),
        pltpu.VMEM((_CH,), jnp.int32),
        pltpu.VMEM((_CH, _D), jnp.float32),
        pltpu.VMEM((_CH, _D), jnp.float32),
        pltpu.VMEM((_CH, _D), jnp.float32),
        pltpu.VMEM((_CH, _D), jnp.float32),
        pltpu.SemaphoreType.DMA,
        pltpu.SemaphoreType.DMA,
        pltpu.SemaphoreType.DMA,
        pltpu.SemaphoreType.DMA,
        pltpu.SemaphoreType.DMA,
        pltpu.SemaphoreType.DMA,
    ],
    compiler_params=pltpu.CompilerParams(use_tc_tiling_on_sc=False),
)(_body)


@jax.jit
def kernel(sequence, segment_labels, tok_table, seg_table):
    comb = (seg_table[:, None, :] + jnp.asarray(_PE)[None, :, :])
    comb = comb.reshape(_NSEG * _SEQ, _D)
    tok128 = _pad_rows(tok_table.T)
    tok2m = tok128.reshape(2 * _VOCAB, _D)
    seq_flat = sequence.reshape(_B)
    seg_flat = segment_labels.reshape(_B)
    out = _lookup(seq_flat, seg_flat, tok2m, comb)
    return out.reshape(_BATCH, _SEQ, _D)


# speedup vs baseline: 1.5213x; 1.0360x over previous
import functools

import numpy as np
import jax
import jax.numpy as jnp
from jax import lax
from jax.experimental import pallas as pl
from jax.experimental.pallas import tpu as pltpu
from jax.experimental.pallas import tpu_sc as plsc

_VOCAB = 1000000
_D = 64
_DP = 128
_NSEG = 3
_SEQ = 200
_BATCH = 1024
_B = _BATCH * _SEQ
_NC, _NS = 2, 16
_NW = _NC * _NS
_BPW = _B // _NW
_CH = 400
_NCHUNK = _BPW // _CH
_LANES = 16
_CB = 16384


def _pad_body(tok_t_ref, out_ref):
    out_ref[:, 0:_D] = tok_t_ref[...].T


_pad_rows = pl.pallas_call(
    _pad_body,
    out_shape=jax.ShapeDtypeStruct((_VOCAB, _DP), jnp.float32),
    grid=((_VOCAB + _CB - 1) // _CB,),
    in_specs=[pl.BlockSpec((_D, _CB), lambda i: (0, i))],
    out_specs=pl.BlockSpec((_CB, _DP), lambda i: (i, 0)),
)


def _pe_table():
    position = np.arange(_SEQ, dtype=np.float32)[:, None]
    div_term = np.exp(
        np.arange(0, _D, 2, dtype=np.float32) * -(np.log(10000.0) / _D))
    pe = np.zeros((_SEQ, _D), dtype=np.float32)
    pe[:, 0::2] = np.sin(position * div_term)
    pe[:, 1::2] = np.cos(position * div_term)
    return pe


_PE = _pe_table()


def _body(seq_hbm, seg_hbm, tok_hbm, comb_hbm, out_hbm,
          idx_v0, idx_v1, seg_v0, seg_v1, cidx_v0, cidx_v1,
          rows_t0, rows_t1, rows_c0, rows_c1,
          sem_t0, sem_t1, sem_c0, sem_c1, sem_o0, sem_o1):
    idx_v = (idx_v0, idx_v1)
    seg_v = (seg_v0, seg_v1)
    cidx_v = (cidx_v0, cidx_v1)
    rows_t = (rows_t0, rows_t1)
    rows_c = (rows_c0, rows_c1)
    sem_t = (sem_t0, sem_t1)
    sem_c = (sem_c0, sem_c1)
    sem_o = (sem_o0, sem_o1)

    wid = lax.axis_index("s") * _NC + lax.axis_index("c")
    base = wid * _BPW

    def chunk_off(c):
        return pl.multiple_of(base + c * _CH, 8)

    def prep(c, s):
        cb = chunk_off(c)
        pltpu.sync_copy(seq_hbm.at[pl.ds(cb, _CH)], idx_v[s])
        pltpu.sync_copy(seg_hbm.at[pl.ds(cb, _CH)], seg_v[s])
        for j in range(_CH // _LANES):
            sl = pl.ds(j * _LANES, _LANES)
            pos = jnp.remainder(
                c * _CH + j * _LANES + lax.iota(jnp.int32, _LANES), _SEQ)
            cidx_v[s][sl] = seg_v[s][sl] * _SEQ + pos
            idx_v[s][sl] = idx_v[s][sl] * 2

    def issue_gathers(s):
        pltpu.async_copy(tok_hbm.at[idx_v[s]], rows_t[s], sem_t[s])
        pltpu.async_copy(comb_hbm.at[cidx_v[s]], rows_c[s], sem_c[s])

    def wait_gathers(s):
        pltpu.make_async_copy(tok_hbm.at[idx_v[s]], rows_t[s], sem_t[s]).wait()
        pltpu.make_async_copy(
            comb_hbm.at[cidx_v[s]], rows_c[s], sem_c[s]).wait()

    def out_desc(c, s):
        return pltpu.make_async_copy(
            rows_c[s], out_hbm.at[pl.ds(chunk_off(c), _CH)], sem_o[s])

    for s in range(2):
        prep(s, s)
        issue_gathers(s)

    def step(i, carry):
        for s in range(2):
            c = 2 * i + s
            wait_gathers(s)

            def add_body(t, acc):
                for j in range(_D // _LANES):
                    sl = pl.ds(j * _LANES, _LANES)
                    rows_c[s][t, sl] = rows_c[s][t, sl] + rows_t[s][t, sl]
                return acc

            lax.fori_loop(0, _CH, add_body, 0, unroll=4)
            out_desc(c, s).start()

            @pl.when(c + 2 < _NCHUNK)
            def _():
                prep(c + 2, s)
                out_desc(c, s).wait()
                issue_gathers(s)

        return carry

    lax.fori_loop(0, _NCHUNK // 2, step, 0)

    for s in range(2):
        out_desc(_NCHUNK - 2 + s, s).wait()


_lookup = functools.partial(
    pl.kernel,
    out_type=jax.ShapeDtypeStruct((_B, _D), jnp.float32),
    mesh=plsc.VectorSubcoreMesh(
        core_axis_name="c", subcore_axis_name="s",
        num_cores=_NC, num_subcores=_NS),
    scratch_types=[
        pltpu.VMEM((_CH,), jnp.int32),
        pltpu.VMEM((_CH,), jnp.int32),
        pltpu.VMEM((_CH,), jnp.int32),
        pltpu.VMEM((_CH,), jnp.int32),
        pltpu.VMEM((_CH,), jnp.int32),
        pltpu.VMEM((_CH,), jnp.int32),
        pltpu.VMEM((_CH, _D), jnp.float32),
        pltpu.VMEM((_CH, _D), jnp.float32),
        pltpu.VMEM((_CH, _D), jnp.float32),
        pltpu.VMEM((_CH, _D), jnp.float32),
        pltpu.SemaphoreType.DMA,
        pltpu.SemaphoreType.DMA,
        pltpu.SemaphoreType.DMA,
        pltpu.SemaphoreType.DMA,
        pltpu.SemaphoreType.DMA,
        pltpu.SemaphoreType.DMA,
    ],
    compiler_params=pltpu.CompilerParams(use_tc_tiling_on_sc=False),
)(_body)


@jax.jit
def kernel(sequence, segment_labels, tok_table, seg_table):
    comb = (seg_table[:, None, :] + jnp.asarray(_PE)[None, :, :])
    comb = comb.reshape(_NSEG * _SEQ, _D)
    tok128 = _pad_rows(tok_table.T)
    tok2m = tok128.reshape(2 * _VOCAB, _D)
    seq_flat = sequence.reshape(_B)
    seg_flat = segment_labels.reshape(_B)
    out = _lookup(seq_flat, seg_flat, tok2m, comb)
    return out.reshape(_BATCH, _SEQ, _D)


# speedup vs baseline: 1.5405x; 1.0126x over previous
import functools

import numpy as np
import jax
import jax.numpy as jnp
from jax import lax
from jax.experimental import pallas as pl
from jax.experimental.pallas import tpu as pltpu
from jax.experimental.pallas import tpu_sc as plsc

_VOCAB = 1000000
_D = 64
_DP = 128
_NSEG = 3
_SEQ = 200
_BATCH = 1024
_B = _BATCH * _SEQ
_NC, _NS = 2, 16
_NW = _NC * _NS
_BPW = _B // _NW
_CH = 400
_NCHUNK = _BPW // _CH
_LANES = 16
_CB = 32768


def _pad_body(tok_t_ref, out_ref):
    out_ref[:, 0:_D] = tok_t_ref[...].T


_pad_rows = pl.pallas_call(
    _pad_body,
    out_shape=jax.ShapeDtypeStruct((_VOCAB, _DP), jnp.float32),
    grid=((_VOCAB + _CB - 1) // _CB,),
    in_specs=[pl.BlockSpec((_D, _CB), lambda i: (0, i))],
    out_specs=pl.BlockSpec((_CB, _DP), lambda i: (i, 0)),
)


def _pe_table():
    position = np.arange(_SEQ, dtype=np.float32)[:, None]
    div_term = np.exp(
        np.arange(0, _D, 2, dtype=np.float32) * -(np.log(10000.0) / _D))
    pe = np.zeros((_SEQ, _D), dtype=np.float32)
    pe[:, 0::2] = np.sin(position * div_term)
    pe[:, 1::2] = np.cos(position * div_term)
    return pe


_PE = _pe_table()


def _body(seq_hbm, seg_hbm, tok_hbm, comb_hbm, out_hbm,
          idx_v0, idx_v1, seg_v0, seg_v1, cidx_v0, cidx_v1,
          rows_t0, rows_t1, rows_c0, rows_c1,
          sem_t0, sem_t1, sem_c0, sem_c1, sem_o0, sem_o1):
    idx_v = (idx_v0, idx_v1)
    seg_v = (seg_v0, seg_v1)
    cidx_v = (cidx_v0, cidx_v1)
    rows_t = (rows_t0, rows_t1)
    rows_c = (rows_c0, rows_c1)
    sem_t = (sem_t0, sem_t1)
    sem_c = (sem_c0, sem_c1)
    sem_o = (sem_o0, sem_o1)

    wid = lax.axis_index("s") * _NC + lax.axis_index("c")
    base = wid * _BPW

    def chunk_off(c):
        return pl.multiple_of(base + c * _CH, 8)

    def prep(c, s):
        cb = chunk_off(c)
        pltpu.sync_copy(seq_hbm.at[pl.ds(cb, _CH)], idx_v[s])
        pltpu.sync_copy(seg_hbm.at[pl.ds(cb, _CH)], seg_v[s])
        for j in range(_CH // _LANES):
            sl = pl.ds(j * _LANES, _LANES)
            pos = jnp.remainder(
                c * _CH + j * _LANES + lax.iota(jnp.int32, _LANES), _SEQ)
            cidx_v[s][sl] = seg_v[s][sl] * _SEQ + pos
            idx_v[s][sl] = idx_v[s][sl] * 2

    def issue_gathers(s):
        pltpu.async_copy(tok_hbm.at[idx_v[s]], rows_t[s], sem_t[s])
        pltpu.async_copy(comb_hbm.at[cidx_v[s]], rows_c[s], sem_c[s])

    def wait_gathers(s):
        pltpu.make_async_copy(tok_hbm.at[idx_v[s]], rows_t[s], sem_t[s]).wait()
        pltpu.make_async_copy(
            comb_hbm.at[cidx_v[s]], rows_c[s], sem_c[s]).wait()

    def out_desc(c, s):
        return pltpu.make_async_copy(
            rows_c[s], out_hbm.at[pl.ds(chunk_off(c), _CH)], sem_o[s])

    for s in range(2):
        prep(s, s)
        issue_gathers(s)

    def step(i, carry):
        for s in range(2):
            c = 2 * i + s
            wait_gathers(s)

            def add_body(t, acc):
                for j in range(_D // _LANES):
                    sl = pl.ds(j * _LANES, _LANES)
                    rows_c[s][t, sl] = rows_c[s][t, sl] + rows_t[s][t, sl]
                return acc

            lax.fori_loop(0, _CH, add_body, 0, unroll=4)
            out_desc(c, s).start()

            @pl.when(c + 2 < _NCHUNK)
            def _():
                prep(c + 2, s)
                out_desc(c, s).wait()
                issue_gathers(s)

        return carry

    lax.fori_loop(0, _NCHUNK // 2, step, 0)

    for s in range(2):
        out_desc(_NCHUNK - 2 + s, s).wait()


_lookup = functools.partial(
    pl.kernel,
    out_type=jax.ShapeDtypeStruct((_B, _D), jnp.float32),
    mesh=plsc.VectorSubcoreMesh(
        core_axis_name="c", subcore_axis_name="s",
        num_cores=_NC, num_subcores=_NS),
    scratch_types=[
        pltpu.VMEM((_CH,), jnp.int32),
        pltpu.VMEM((_CH,), jnp.int32),
        pltpu.VMEM((_CH,), jnp.int32),
        pltpu.VMEM((_CH,), jnp.int32),
        pltpu.VMEM((_CH,), jnp.int32),
        pltpu.VMEM((_CH,), jnp.int32),
        pltpu.VMEM((_CH, _D), jnp.float32),
        pltpu.VMEM((_CH, _D), jnp.float32),
        pltpu.VMEM((_CH, _D), jnp.float32),
        pltpu.VMEM((_CH, _D), jnp.float32),
        pltpu.SemaphoreType.DMA,
        pltpu.SemaphoreType.DMA,
        pltpu.SemaphoreType.DMA,
        pltpu.SemaphoreType.DMA,
        pltpu.SemaphoreType.DMA,
        pltpu.SemaphoreType.DMA,
    ],
    compiler_params=pltpu.CompilerParams(use_tc_tiling_on_sc=False),
)(_body)


@jax.jit
def kernel(sequence, segment_labels, tok_table, seg_table):
    comb = (seg_table[:, None, :] + jnp.asarray(_PE)[None, :, :])
    comb = comb.reshape(_NSEG * _SEQ, _D)
    tok128 = _pad_rows(tok_table.T)
    tok2m = tok128.reshape(2 * _VOCAB, _D)
    seq_flat = sequence.reshape(_B)
    seg_flat = segment_labels.reshape(_B)
    out = _lookup(seq_flat, seg_flat, tok2m, comb)
    return out.reshape(_BATCH, _SEQ, _D)


# speedup vs baseline: 1.7529x; 1.1379x over previous
import functools

import numpy as np
import jax
import jax.numpy as jnp
from jax import lax
from jax.experimental import pallas as pl
from jax.experimental.pallas import tpu as pltpu
from jax.experimental.pallas import tpu_sc as plsc

_VOCAB = 1000000
_D = 64
_DP = 128
_NSEG = 3
_SEQ = 200
_BATCH = 1024
_B = _BATCH * _SEQ
_NC, _NS = 2, 16
_NW = _NC * _NS
_BPW = _B // _NW
_CH = 320
_NCHUNK = _BPW // _CH
_LANES = 16
_CB = 32768


def _pad_body(tok_t_ref, out_ref):
    out_ref[:, 0:_D] = tok_t_ref[...].T


_pad_rows = pl.pallas_call(
    _pad_body,
    out_shape=jax.ShapeDtypeStruct((_VOCAB, _DP), jnp.float32),
    grid=((_VOCAB + _CB - 1) // _CB,),
    in_specs=[pl.BlockSpec((_D, _CB), lambda i: (0, i))],
    out_specs=pl.BlockSpec((_CB, _DP), lambda i: (i, 0)),
)


def _pe_table():
    position = np.arange(_SEQ, dtype=np.float32)[:, None]
    div_term = np.exp(
        np.arange(0, _D, 2, dtype=np.float32) * -(np.log(10000.0) / _D))
    pe = np.zeros((_SEQ, _D), dtype=np.float32)
    pe[:, 0::2] = np.sin(position * div_term)
    pe[:, 1::2] = np.cos(position * div_term)
    return pe


_PE = _pe_table()


def _body(seq_hbm, seg_hbm, tok_hbm, comb_hbm, out_hbm,
          idx_v0, idx_v1, seg_v0, seg_v1, cidx_v0, cidx_v1,
          rows_t0, rows_t1, rows_c0, rows_c1,
          sem_t0, sem_t1, sem_c0, sem_c1, sem_o0, sem_o1):
    idx_v = (idx_v0, idx_v1)
    seg_v = (seg_v0, seg_v1)
    cidx_v = (cidx_v0, cidx_v1)
    rows_t = (rows_t0, rows_t1)
    rows_c = (rows_c0, rows_c1)
    sem_t = (sem_t0, sem_t1)
    sem_c = (sem_c0, sem_c1)
    sem_o = (sem_o0, sem_o1)

    wid = lax.axis_index("s") * _NC + lax.axis_index("c")
    base = wid * _BPW

    def chunk_off(c):
        return pl.multiple_of(base + c * _CH, 8)

    def prep(c, s):
        cb = chunk_off(c)
        pltpu.sync_copy(seq_hbm.at[pl.ds(cb, _CH)], idx_v[s])
        pltpu.sync_copy(seg_hbm.at[pl.ds(cb, _CH)], seg_v[s])
        for j in range(_CH // _LANES):
            sl = pl.ds(j * _LANES, _LANES)
            pos = jnp.remainder(
                c * _CH + j * _LANES + lax.iota(jnp.int32, _LANES), _SEQ)
            cidx_v[s][sl] = seg_v[s][sl] * _SEQ + pos
            idx_v[s][sl] = idx_v[s][sl] * 2

    def issue_gathers(s):
        pltpu.async_copy(tok_hbm.at[idx_v[s]], rows_t[s], sem_t[s])
        pltpu.async_copy(comb_hbm.at[cidx_v[s]], rows_c[s], sem_c[s])

    def wait_gathers(s):
        pltpu.make_async_copy(tok_hbm.at[idx_v[s]], rows_t[s], sem_t[s]).wait()
        pltpu.make_async_copy(
            comb_hbm.at[cidx_v[s]], rows_c[s], sem_c[s]).wait()

    def out_desc(c, s):
        return pltpu.make_async_copy(
            rows_c[s], out_hbm.at[pl.ds(chunk_off(c), _CH)], sem_o[s])

    for s in range(2):
        prep(s, s)
        issue_gathers(s)

    def step(i, carry):
        for s in range(2):
            c = 2 * i + s
            wait_gathers(s)

            def add_body(t, acc):
                for j in range(_D // _LANES):
                    sl = pl.ds(j * _LANES, _LANES)
                    rows_c[s][t, sl] = rows_c[s][t, sl] + rows_t[s][t, sl]
                return acc

            lax.fori_loop(0, _CH, add_body, 0, unroll=4)
            out_desc(c, s).start()

            @pl.when(c + 2 < _NCHUNK)
            def _():
                prep(c + 2, s)
                out_desc(c, s).wait()
                issue_gathers(s)

        return carry

    lax.fori_loop(0, _NCHUNK // 2, step, 0)

    for s in range(2):
        out_desc(_NCHUNK - 2 + s, s).wait()


_lookup = functools.partial(
    pl.kernel,
    out_type=jax.ShapeDtypeStruct((_B, _DP), jnp.float32),
    mesh=plsc.VectorSubcoreMesh(
        core_axis_name="c", subcore_axis_name="s",
        num_cores=_NC, num_subcores=_NS),
    scratch_types=[
        pltpu.VMEM((_CH,), jnp.int32),
        pltpu.VMEM((_CH,), jnp.int32),
        pltpu.VMEM((_CH,), jnp.int32),
        pltpu.VMEM((_CH,), jnp.int32),
        pltpu.VMEM((_CH,), jnp.int32),
        pltpu.VMEM((_CH,), jnp.int32),
        pltpu.VMEM((_CH, _D), jnp.float32),
        pltpu.VMEM((_CH, _D), jnp.float32),
        pltpu.VMEM((_CH, _DP), jnp.float32),
        pltpu.VMEM((_CH, _DP), jnp.float32),
        pltpu.SemaphoreType.DMA,
        pltpu.SemaphoreType.DMA,
        pltpu.SemaphoreType.DMA,
        pltpu.SemaphoreType.DMA,
        pltpu.SemaphoreType.DMA,
        pltpu.SemaphoreType.DMA,
    ],
    compiler_params=pltpu.CompilerParams(use_tc_tiling_on_sc=False),
)(_body)


@jax.jit
def kernel(sequence, segment_labels, tok_table, seg_table):
    comb = (seg_table[:, None, :] + jnp.asarray(_PE)[None, :, :])
    comb = comb.reshape(_NSEG * _SEQ, _D)
    comb = jnp.concatenate(
        [comb, jnp.zeros((_NSEG * _SEQ, _DP - _D), jnp.float32)], axis=1)
    tok128 = _pad_rows(tok_table.T)
    tok2m = tok128.reshape(2 * _VOCAB, _D)
    seq_flat = sequence.reshape(_B)
    seg_flat = segment_labels.reshape(_B)
    out = _lookup(seq_flat, seg_flat, tok2m, comb)
    return out[:, :_D].reshape(_BATCH, _SEQ, _D)
